# baseline (speedup 1.0000x reference)
import functools

import jax
import jax.numpy as jnp
from jax import lax
from jax.experimental import pallas as pl
from jax.experimental.pallas import tpu as pltpu
from jax.experimental.pallas import tpu_sc as plsc

_EMB = 32
_ROW = 128
_IDXW = 128
_L = 16


@functools.lru_cache(maxsize=None)
def _build_gather(n_slab, batch):
    nc, ns = 2, 16
    nw = nc * ns
    slabs_per_w = n_slab // nw
    kb = batch // _IDXW
    assert n_slab % (2 * nw) == 0 and kb == 2

    def body(node_hbm, trans_hbm, idx_hbm, out_hbm,
             idxb0, idxb1, gn0, gn1, gt0, gt1, comb0, comb1,
             semg0, semg1, semw0, semw1, semi0, semi1):
        idxb, gn, gt = (idxb0, idxb1), (gn0, gn1), (gt0, gt1)
        comb, semg, semw = (comb0, comb1), (semg0, semg1), (semw0, semw1)
        semi = (semi0, semi1)
        wid = lax.axis_index("s") * nc + lax.axis_index("c")
        s0 = wid * slabs_per_w
        iota = lax.iota(jnp.int32, _L)

        def fire(s, islot, h, gslot):
            pltpu.async_copy(node_hbm.at[idxb[islot].at[h]], gn[gslot],
                             semg[gslot])
            pltpu.async_copy(trans_hbm.at[idxb[islot].at[2 + h]], gt[gslot],
                             semg[gslot])

        def wait_g(islot, h, gslot, t):
            if t == 0:
                pltpu.make_async_copy(node_hbm.at[idxb[islot].at[h]],
                                      gn[gslot], semg[gslot]).wait()
            else:
                pltpu.make_async_copy(trans_hbm.at[idxb[islot].at[2 + h]],
                                      gt[gslot], semg[gslot]).wait()

        def assemble(islot, h, sp, t):
            def grp(gi, carry2):
                lrows = gi * _L + iota
                ids = idxb[islot][4 + 2 * t + h, pl.ds(gi * _L, _L)]
                q32 = (ids & 3) << 5
                src = (gn, gt)[t][h]
                ocol = h * _IDXW + gi * _L + iota
                for e0 in range(_EMB):
                    m = (e0 + iota) & (_EMB - 1)
                    v = plsc.load_gather(src, [lrows, q32 + m])
                    plsc.store_scatter(comb[sp], [m + t * _EMB, ocol], v)
                return carry2

            lax.fori_loop(0, _IDXW // _L, grp, 0)

        pltpu.sync_copy(idx_hbm.at[s0], idxb[0])
        fire(s0, 0, 0, 0)

        def pair(gg, carry):
            for sp in (0, 1):
                sl = gg * 2 + sp
                s = s0 + sl

                @pl.when(sl >= 1)
                def _():
                    pltpu.make_async_copy(
                        idx_hbm.at[s], idxb[sp], semi[sp]).wait()

                @pl.when(sl < slabs_per_w - 1)
                def _():
                    pltpu.async_copy(
                        idx_hbm.at[s + 1], idxb[(sp + 1) % 2],
                        semi[(sp + 1) % 2])

                @pl.when(sl >= 2)
                def _():
                    pltpu.make_async_copy(
                        comb[sp], out_hbm.at[s0 + sp], semw[sp]).wait()

                for h in (0, 1):
                    if h == 0:
                        fire(s, sp, 1, 1)
                    else:
                        @pl.when(sl < slabs_per_w - 1)
                        def _():
                            fire(s + 1, (sp + 1) % 2, 0, 0)
                    wait_g(sp, h, h, 0)
                    assemble(sp, h, sp, 0)
                    wait_g(sp, h, h, 1)
                    assemble(sp, h, sp, 1)

                pltpu.async_copy(comb[sp], out_hbm.at[s], semw[sp])
            return carry

        lax.fori_loop(0, slabs_per_w // 2, pair, 0)
        pltpu.make_async_copy(comb[0], out_hbm.at[s0], semw[0]).wait()
        pltpu.make_async_copy(comb[1], out_hbm.at[s0], semw[1]).wait()

    return pl.kernel(
        body,
        mesh=plsc.VectorSubcoreMesh(core_axis_name="c", subcore_axis_name="s"),
        compiler_params=pltpu.CompilerParams(needs_layout_passes=False),
        out_type=jax.ShapeDtypeStruct((n_slab, 2 * _EMB, batch), jnp.float32),
        scratch_types=[
            pltpu.VMEM((4 * 2, _IDXW), jnp.int32),
            pltpu.VMEM((4 * 2, _IDXW), jnp.int32),
            pltpu.VMEM((_IDXW, _ROW), jnp.float32),
            pltpu.VMEM((_IDXW, _ROW), jnp.float32),
            pltpu.VMEM((_IDXW, _ROW), jnp.float32),
            pltpu.VMEM((_IDXW, _ROW), jnp.float32),
            pltpu.VMEM((2 * _EMB, batch), jnp.float32),
            pltpu.VMEM((2 * _EMB, batch), jnp.float32),
            pltpu.SemaphoreType.DMA,
            pltpu.SemaphoreType.DMA,
            pltpu.SemaphoreType.DMA,
            pltpu.SemaphoreType.DMA,
            pltpu.SemaphoreType.DMA,
            pltpu.SemaphoreType.DMA,
        ],
    )


def _pack(table):
    rows = (table.shape[0] * _EMB) // _ROW
    return table.reshape(-1)[: rows * _ROW].reshape(rows, _ROW)


def kernel(input_X, input_A, node_table, transition_table, max_batch_length):
    b, mbl, n, _ = input_X.shape
    n_slab = mbl * n
    kb = b // _IDXW
    delta = jnp.asarray(max_batch_length).astype(jnp.int32) - mbl
    flat_x = input_X.reshape(b, n_slab, input_X.shape[-1]).astype(jnp.int32)
    nidf = flat_x[:, :, 1] + delta
    tidf = flat_x[:, :, 4] + delta
    nidf, tidf = lax.optimization_barrier((nidf, tidf))
    nid = nidf.T.reshape(n_slab, kb, _IDXW)
    tid = tidf.T.reshape(n_slab, kb, _IDXW)
    idx_all = jnp.concatenate(
        [nid >> 2, tid >> 2, nid, tid], axis=1)
    out = _build_gather(n_slab, b)(_pack(node_table), _pack(transition_table),
                                   idx_all)
    emb = out.reshape(mbl, n, 2 * _EMB, b).transpose(3, 0, 1, 2)
    return (emb, input_A.astype(jnp.float32))
